# Initial kernel scaffold; baseline (speedup 1.0000x reference)
import jax
import jax.numpy as jnp
from jax.experimental import pallas as pl


def kernel(pred, target, x, pos, edge_index, edge_attr, external_force):
    raise NotImplementedError("write your pallas kernel here")


# speedup vs baseline: 29.0938x; 29.0938x over previous
import jax
import jax.numpy as jnp
from jax import lax
from jax.experimental import pallas as pl
from jax.experimental.pallas import tpu as pltpu
from jax.experimental.pallas import tpu_sc as plsc

_MU = 0.001
_RHO = 1000.0
_DT = 1e-05
_WEIGHT = 1.0

_NC = 2
_NS = 16
_NW = _NC * _NS
_L = 16
_IB = 64
_KA = 512
_KB = 1024

_CP = pltpu.CompilerParams(needs_layout_passes=False, use_tc_tiling_on_sc=False)


def _mesh():
    return plsc.VectorSubcoreMesh(core_axis_name="c", subcore_axis_name="s")


def _iota16():
    return lax.iota(jnp.int32, _L)


def _col(j):
    return jnp.full((_L,), j, jnp.int32)


def _sub_rows(total, cap):
    for d in range(cap, 15, -16):
        if total % d == 0:
            return d
    raise ValueError(f"no row subchunk for {total}")


def _pass_a(pv8, ei3, attr, zeros16, n_nodes_p, n_edges):
    rpt = n_nodes_p // _NS
    rz = _sub_rows(rpt, 2048)
    ew = n_edges // _NW
    k = _KA
    nib = k // _IB
    nchunk = ew // k
    ngrp = k // _L

    def body(pv_h, ei3_h, attr_h, z16_h, gp_h,
             g_sp, src_v, dst_v, attr_v, pvs_v, pvd_v, con_v):
        c = lax.axis_index("c")
        s = lax.axis_index("s")
        wid = c * _NS + s
        r0 = s * rpt
        lanes = _iota16()

        @pl.loop(0, rpt // rz)
        def _(m):
            pltpu.sync_copy(z16_h.at[pl.ds(0, rz)],
                            g_sp.at[pl.ds(r0 + m * rz, rz)])

        zeros = jnp.zeros((_L,), jnp.float32)
        ones = jnp.full((_L,), 1.0, jnp.float32)

        @pl.loop(0, ngrp)
        def _(g):
            rows = g * _L + lanes
            plsc.store_scatter(con_v, [rows, _col(9)], ones)
            for j in range(10, 16):
                plsc.store_scatter(con_v, [rows, _col(j)], zeros)

        plsc.subcore_barrier()

        @pl.loop(0, nchunk)
        def _(ch):
            ebase = wid * ew + ch * k
            rbase = wid * (ew // _IB) + ch * nib
            pltpu.sync_copy(ei3_h.at[0, pl.ds(rbase, nib)], src_v)
            pltpu.sync_copy(ei3_h.at[1, pl.ds(rbase, nib)], dst_v)
            pltpu.sync_copy(attr_h.at[pl.ds(ebase, k)], attr_v)

            @pl.loop(0, nib)
            def _(j):
                pltpu.sync_copy(pv_h.at[src_v.at[j]],
                                pvs_v.at[pl.ds(j * _IB, _IB)])
                pltpu.sync_copy(pv_h.at[dst_v.at[j]],
                                pvd_v.at[pl.ds(j * _IB, _IB)])

            @pl.loop(0, ngrp)
            def _(g):
                rows = g * _L + lanes
                ps = [plsc.load_gather(pvs_v, [rows, _col(j)]) for j in range(6)]
                pd = [plsc.load_gather(pvd_v, [rows, _col(j)]) for j in range(6)]
                w = plsc.load_gather(attr_v, [rows, _col(0)])
                d = [pd[j] - ps[j] for j in range(3)]
                dist2 = d[0] * d[0] + d[1] * d[1] + d[2] * d[2] + 1e-8
                rcp = 1.0 / dist2
                e = [dj * rcp for dj in d]
                t = [w * (pd[3 + i] - ps[3 + i]) for i in range(3)]
                for i in range(3):
                    for j in range(3):
                        plsc.store_scatter(con_v, [rows, _col(3 * i + j)],
                                           t[i] * e[j])

            @pl.loop(0, nib)
            def _(j):
                pltpu.sync_copy(con_v.at[pl.ds(j * _IB, _IB)],
                                g_sp.at[src_v.at[j]], add=True)

        plsc.subcore_barrier()

        @pl.loop(0, rpt // rz)
        def _(m):
            pltpu.sync_copy(g_sp.at[pl.ds(r0 + m * rz, rz)],
                            gp_h.at[c, pl.ds(r0 + m * rz, rz)])

    return pl.kernel(
        body,
        compiler_params=_CP,
        out_type=jax.ShapeDtypeStruct((_NC, n_nodes_p, 16), jnp.float32),
        mesh=_mesh(),
        scratch_types=[
            pltpu.VMEM_SHARED((n_nodes_p, 16), jnp.float32),
            pltpu.VMEM((nib, _IB), jnp.int32),
            pltpu.VMEM((nib, _IB), jnp.int32),
            pltpu.VMEM((k, 4), jnp.float32),
            pltpu.VMEM((k, 8), jnp.float32),
            pltpu.VMEM((k, 8), jnp.float32),
            pltpu.VMEM((k, 16), jnp.float32),
        ],
        name="mcl_grad_pass",
    )(pv8, ei3, attr, zeros16)


def _combine(pv8, gp, n_nodes_p):
    wpr = n_nodes_p // _NW
    r = _sub_rows(wpr, 512)

    def body(pv_h, gp_h, t_h, g0_v, g1_v, pv_v, t_v):
        c = lax.axis_index("c")
        s = lax.axis_index("s")
        wid = c * _NS + s
        lanes = _iota16()
        zeros = jnp.zeros((_L,), jnp.float32)

        @pl.loop(0, wpr // r)
        def _(m):
            r0 = wid * wpr + m * r
            pltpu.sync_copy(gp_h.at[0, pl.ds(r0, r)], g0_v)
            pltpu.sync_copy(gp_h.at[1, pl.ds(r0, r)], g1_v)
            pltpu.sync_copy(pv_h.at[pl.ds(r0, r)], pv_v)

            @pl.loop(0, r // _L)
            def _(g):
                rows = g * _L + lanes
                deg = (plsc.load_gather(g0_v, [rows, _col(9)]) +
                       plsc.load_gather(g1_v, [rows, _col(9)]))
                degi = 1.0 / jnp.maximum(deg, 1.0)
                for j in range(3):
                    plsc.store_scatter(
                        t_v, [rows, _col(j)],
                        plsc.load_gather(pv_v, [rows, _col(j)]))
                for q in range(9):
                    gv = (plsc.load_gather(g0_v, [rows, _col(q)]) +
                          plsc.load_gather(g1_v, [rows, _col(q)])) * degi
                    plsc.store_scatter(t_v, [rows, _col(3 + q)], gv)
                plsc.store_scatter(t_v, [rows, _col(12)], degi)
                for j in range(13, 16):
                    plsc.store_scatter(t_v, [rows, _col(j)], zeros)

            pltpu.sync_copy(t_v, t_h.at[pl.ds(r0, r)])

    return pl.kernel(
        body,
        compiler_params=_CP,
        out_type=jax.ShapeDtypeStruct((n_nodes_p, 16), jnp.float32),
        mesh=_mesh(),
        scratch_types=[
            pltpu.VMEM((r, 16), jnp.float32),
            pltpu.VMEM((r, 16), jnp.float32),
            pltpu.VMEM((r, 8), jnp.float32),
            pltpu.VMEM((r, 16), jnp.float32),
        ],
        name="mcl_combine",
    )(pv8, gp)


def _pass_b(t16, ei3, zeros8, n_nodes_p, n_edges):
    rpt = n_nodes_p // _NS
    rz = _sub_rows(rpt, 2048)
    ew = n_edges // _NW
    k = _KB
    nib = k // _IB
    nchunk = ew // k
    ngrp = k // _L

    def body(t_h, ei3_h, z8_h, dp_h,
             d_sp, src_v, dst_v, ts_v, td_v, cb_v, db_v, tb_v):
        c = lax.axis_index("c")
        s = lax.axis_index("s")
        wid = c * _NS + s
        r0 = s * rpt
        lanes = _iota16()
        zeros = jnp.zeros((_L,), jnp.float32)

        @pl.loop(0, rpt // rz)
        def _(m):
            pltpu.sync_copy(z8_h.at[pl.ds(0, rz)],
                            d_sp.at[pl.ds(r0 + m * rz, rz)])

        @pl.loop(0, ngrp)
        def _(g):
            rows = g * _L + lanes
            for j in range(3, 8):
                plsc.store_scatter(cb_v, [rows, _col(j)], zeros)

        plsc.subcore_barrier()

        @pl.loop(0, nchunk)
        def _(ch):
            rbase = wid * (ew // _IB) + ch * nib
            pltpu.sync_copy(ei3_h.at[0, pl.ds(rbase, nib)], src_v)
            pltpu.sync_copy(ei3_h.at[1, pl.ds(rbase, nib)], dst_v)

            @pl.loop(0, nib)
            def _(j):
                pltpu.sync_copy(t_h.at[src_v.at[j]],
                                ts_v.at[pl.ds(j * _IB, _IB)])
                pltpu.sync_copy(t_h.at[dst_v.at[j]],
                                td_v.at[pl.ds(j * _IB, _IB)])

            @pl.loop(0, ngrp)
            def _(g):
                rows = g * _L + lanes
                ts = [plsc.load_gather(ts_v, [rows, _col(j)]) for j in range(12)]
                td = [plsc.load_gather(td_v, [rows, _col(j)]) for j in range(12)]
                d = [td[j] - ts[j] for j in range(3)]
                dist2 = d[0] * d[0] + d[1] * d[1] + d[2] * d[2] + 1e-8
                rcp = 1.0 / dist2
                e = [dj * rcp for dj in d]
                for i in range(3):
                    ci = ((td[3 + 3 * i] - ts[3 + 3 * i]) * e[0] +
                          (td[4 + 3 * i] - ts[4 + 3 * i]) * e[1] +
                          (td[5 + 3 * i] - ts[5 + 3 * i]) * e[2])
                    plsc.store_scatter(cb_v, [rows, _col(i)], ci)

            @pl.loop(0, nib)
            def _(j):
                pltpu.sync_copy(cb_v.at[pl.ds(j * _IB, _IB)],
                                d_sp.at[src_v.at[j]], add=True)

        plsc.subcore_barrier()

        @pl.loop(0, rpt // rz)
        def _(m):
            r1 = r0 + m * rz
            pltpu.sync_copy(d_sp.at[pl.ds(r1, rz)], db_v)
            pltpu.sync_copy(t_h.at[pl.ds(r1, rz)], tb_v)

            @pl.loop(0, rz // _L)
            def _(g):
                rows = g * _L + lanes
                degi = plsc.load_gather(tb_v, [rows, _col(12)])
                for j in range(3):
                    v = plsc.load_gather(db_v, [rows, _col(j)]) * degi
                    plsc.store_scatter(db_v, [rows, _col(j)], v)

            pltpu.sync_copy(db_v, dp_h.at[c, pl.ds(r1, rz)])

    return pl.kernel(
        body,
        compiler_params=_CP,
        out_type=jax.ShapeDtypeStruct((_NC, n_nodes_p, 8), jnp.float32),
        mesh=_mesh(),
        scratch_types=[
            pltpu.VMEM_SHARED((n_nodes_p, 8), jnp.float32),
            pltpu.VMEM((nib, _IB), jnp.int32),
            pltpu.VMEM((nib, _IB), jnp.int32),
            pltpu.VMEM((k, 16), jnp.float32),
            pltpu.VMEM((k, 16), jnp.float32),
            pltpu.VMEM((k, 8), jnp.float32),
            pltpu.VMEM((rz, 8), jnp.float32),
            pltpu.VMEM((rz, 16), jnp.float32),
        ],
        name="mcl_div_pass",
    )(t16, ei3, zeros8)


def _loss_tc(a, b, c0, c1, n_valid):
    def body(a_ref, b_ref, c0_ref, c1_ref, o_ref):
        lap = c0_ref[...] + c1_ref[...]
        phys = ((_MU / _RHO) * lap + b_ref[...] * (1.0 / _RHO)) * _DT
        res = a_ref[...] - phys
        o_ref[0, 0] = _WEIGHT * jnp.sum(res * res) * (1.0 / n_valid)

    out = pl.pallas_call(
        body,
        out_shape=jax.ShapeDtypeStruct((1, 1), jnp.float32),
        in_specs=[pl.BlockSpec(memory_space=pltpu.VMEM)] * 4,
        out_specs=pl.BlockSpec(memory_space=pltpu.SMEM),
        name="mcl_loss",
    )(a, b, c0, c1)
    return out[0, 0]


def _flatpad(v, n_flat_p):
    v = v.reshape(-1)
    pad = n_flat_p - v.shape[0]
    if pad:
        v = jnp.concatenate([v, jnp.zeros((pad,), v.dtype)])
    return v.reshape(n_flat_p // 128, 128)


def kernel(pred, target, x, pos, edge_index, edge_attr, external_force):
    n = pos.shape[0]
    n_edges = edge_index.shape[1]

    rpt = ((n + _NS - 1) // _NS + 63) // 64 * 64
    n_p = rpt * _NS

    velocity = x[:, 5:8]
    pv8 = jnp.concatenate(
        [pos, velocity, jnp.zeros((n, 2), jnp.float32)], axis=1)
    pv8 = jnp.concatenate([pv8, jnp.zeros((n_p - n, 8), jnp.float32)], axis=0)

    unit = _NW * max(_KA, _KB)
    e_p = -(-n_edges // unit) * unit
    e_pad = e_p - n_edges
    ei = edge_index.astype(jnp.int32)
    if e_pad:
        ei = jnp.concatenate(
            [ei, jnp.full((2, e_pad), n_p - 1, jnp.int32)], axis=1)
    ei3 = ei.reshape(2, e_p // _IB, _IB)
    attr = edge_attr.astype(jnp.float32)
    if e_pad:
        attr = jnp.concatenate(
            [attr, jnp.zeros((e_pad, attr.shape[1]), jnp.float32)], axis=0)
    zeros16 = jnp.zeros((_sub_rows(rpt, 2048), 16), jnp.float32)
    zeros8 = jnp.zeros((_sub_rows(rpt, 2048), 8), jnp.float32)

    gp = _pass_a(pv8, ei3, attr, zeros16, n_p, e_p)
    t16 = _combine(pv8, gp, n_p)
    dp = _pass_b(t16, ei3, zeros8, n_p, e_p)

    n_flat = 3 * n
    n_flat_p = (n_flat + 127) // 128 * 128
    a = _flatpad(pred[:, 2:5].astype(jnp.float32), n_flat_p)
    b = _flatpad(external_force.astype(jnp.float32), n_flat_p)
    c0 = _flatpad(dp[0, :n, 0:3], n_flat_p)
    c1 = _flatpad(dp[1, :n, 0:3], n_flat_p)
    return _loss_tc(a, b, c0, c1, float(n_flat))


# speedup vs baseline: 55.3708x; 1.9032x over previous
import jax
import jax.numpy as jnp
from jax import lax
from jax.experimental import pallas as pl
from jax.experimental.pallas import tpu as pltpu
from jax.experimental.pallas import tpu_sc as plsc

_MU = 0.001
_RHO = 1000.0
_DT = 1e-05
_WEIGHT = 1.0

_NC = 2
_NS = 16
_NW = _NC * _NS
_L = 16
_IB = 64
_KA = 512
_KB = 1024

_CP = pltpu.CompilerParams(needs_layout_passes=False, use_tc_tiling_on_sc=False)


def _mesh():
    return plsc.VectorSubcoreMesh(core_axis_name="c", subcore_axis_name="s")


def _iota16():
    return lax.iota(jnp.int32, _L)


def _col(j):
    return jnp.full((_L,), j, jnp.int32)


def _sub_rows(total, cap):
    for d in range(cap, 15, -16):
        if total % d == 0:
            return d
    raise ValueError(f"no row subchunk for {total}")


def _pass_a(pv8, src, dst, wgt, zeros16, n_nodes_p, n_edges):
    rpt = n_nodes_p // _NS
    rz = _sub_rows(rpt, 2048)
    ew = n_edges // _NW
    k = _KA
    nchunk = ew // k
    tail = ew - nchunk * k
    assert tail % _IB == 0

    def body(pv_h, src_h, dst_h, w_h, z16_h, gp_h,
             g_sp, src_v, dst_v, w_v, pvs_v, pvd_v, con_v):
        c = lax.axis_index("c")
        s = lax.axis_index("s")
        wid = c * _NS + s
        r0 = s * rpt
        lanes = _iota16()

        @pl.loop(0, rpt // rz)
        def _(m):
            pltpu.sync_copy(z16_h.at[pl.ds(0, rz)],
                            g_sp.at[pl.ds(r0 + m * rz, rz)])

        zeros = jnp.zeros((_L,), jnp.float32)
        ones = jnp.full((_L,), 1.0, jnp.float32)

        @pl.loop(0, k // _L)
        def _(g):
            rows = g * _L + lanes
            plsc.store_scatter(con_v, [rows, _col(9)], ones)
            for j in range(10, 16):
                plsc.store_scatter(con_v, [rows, _col(j)], zeros)

        plsc.subcore_barrier()

        def chunk(ebase, kc):
            nib = kc // _IB
            pltpu.sync_copy(src_h.at[pl.ds(ebase, kc)], src_v.at[pl.ds(0, kc)])
            pltpu.sync_copy(dst_h.at[pl.ds(ebase, kc)], dst_v.at[pl.ds(0, kc)])
            pltpu.sync_copy(w_h.at[pl.ds(ebase, kc)], w_v.at[pl.ds(0, kc)])

            @pl.loop(0, nib)
            def _(j):
                pltpu.sync_copy(pv_h.at[src_v.at[pl.ds(j * _IB, _IB)]],
                                pvs_v.at[pl.ds(j * _IB, _IB)])
                pltpu.sync_copy(pv_h.at[dst_v.at[pl.ds(j * _IB, _IB)]],
                                pvd_v.at[pl.ds(j * _IB, _IB)])

            @pl.loop(0, kc // _L)
            def _(g):
                rows = g * _L + lanes
                ps = [plsc.load_gather(pvs_v, [rows, _col(j)]) for j in range(6)]
                pd = [plsc.load_gather(pvd_v, [rows, _col(j)]) for j in range(6)]
                w = w_v[pl.ds(g * _L, _L)]
                d = [pd[j] - ps[j] for j in range(3)]
                dist2 = d[0] * d[0] + d[1] * d[1] + d[2] * d[2] + 1e-8
                rcp = 1.0 / dist2
                e = [dj * rcp for dj in d]
                t = [w * (pd[3 + i] - ps[3 + i]) for i in range(3)]
                for i in range(3):
                    for j in range(3):
                        plsc.store_scatter(con_v, [rows, _col(3 * i + j)],
                                           t[i] * e[j])

            @pl.loop(0, nib)
            def _(j):
                pltpu.sync_copy(con_v.at[pl.ds(j * _IB, _IB)],
                                g_sp.at[src_v.at[pl.ds(j * _IB, _IB)]],
                                add=True)

        @pl.loop(0, nchunk)
        def _(ch):
            chunk(wid * ew + ch * k, k)

        if tail:
            chunk(wid * ew + nchunk * k, tail)

        plsc.subcore_barrier()

        @pl.loop(0, rpt // rz)
        def _(m):
            pltpu.sync_copy(g_sp.at[pl.ds(r0 + m * rz, rz)],
                            gp_h.at[c, pl.ds(r0 + m * rz, rz)])

    return pl.kernel(
        body,
        compiler_params=_CP,
        out_type=jax.ShapeDtypeStruct((_NC, n_nodes_p, 16), jnp.float32),
        mesh=_mesh(),
        scratch_types=[
            pltpu.VMEM_SHARED((n_nodes_p, 16), jnp.float32),
            pltpu.VMEM((k,), jnp.int32),
            pltpu.VMEM((k,), jnp.int32),
            pltpu.VMEM((k,), jnp.float32),
            pltpu.VMEM((k, 8), jnp.float32),
            pltpu.VMEM((k, 8), jnp.float32),
            pltpu.VMEM((k, 16), jnp.float32),
        ],
        name="mcl_grad_pass",
    )(pv8, src, dst, wgt, zeros16)


def _combine(pv8, gp, n_nodes_p):
    wpr = n_nodes_p // _NW
    r = _sub_rows(wpr, 512)

    def body(pv_h, gp_h, t_h, g0_v, g1_v, pv_v, t_v):
        c = lax.axis_index("c")
        s = lax.axis_index("s")
        wid = c * _NS + s
        lanes = _iota16()
        zeros = jnp.zeros((_L,), jnp.float32)

        @pl.loop(0, wpr // r)
        def _(m):
            r0 = wid * wpr + m * r
            pltpu.sync_copy(gp_h.at[0, pl.ds(r0, r)], g0_v)
            pltpu.sync_copy(gp_h.at[1, pl.ds(r0, r)], g1_v)
            pltpu.sync_copy(pv_h.at[pl.ds(r0, r)], pv_v)

            @pl.loop(0, r // _L)
            def _(g):
                rows = g * _L + lanes
                deg = (plsc.load_gather(g0_v, [rows, _col(9)]) +
                       plsc.load_gather(g1_v, [rows, _col(9)]))
                degi = 1.0 / jnp.maximum(deg, 1.0)
                for j in range(3):
                    plsc.store_scatter(
                        t_v, [rows, _col(j)],
                        plsc.load_gather(pv_v, [rows, _col(j)]))
                for q in range(9):
                    gv = (plsc.load_gather(g0_v, [rows, _col(q)]) +
                          plsc.load_gather(g1_v, [rows, _col(q)])) * degi
                    plsc.store_scatter(t_v, [rows, _col(3 + q)], gv)
                plsc.store_scatter(t_v, [rows, _col(12)], degi)
                for j in range(13, 16):
                    plsc.store_scatter(t_v, [rows, _col(j)], zeros)

            pltpu.sync_copy(t_v, t_h.at[pl.ds(r0, r)])

    return pl.kernel(
        body,
        compiler_params=_CP,
        out_type=jax.ShapeDtypeStruct((n_nodes_p, 16), jnp.float32),
        mesh=_mesh(),
        scratch_types=[
            pltpu.VMEM((r, 16), jnp.float32),
            pltpu.VMEM((r, 16), jnp.float32),
            pltpu.VMEM((r, 8), jnp.float32),
            pltpu.VMEM((r, 16), jnp.float32),
        ],
        name="mcl_combine",
    )(pv8, gp)


def _pass_b(t16, src, dst, zeros8, n_nodes_p, n_edges):
    rpt = n_nodes_p // _NS
    rz = _sub_rows(rpt, 2048)
    ew = n_edges // _NW
    k = _KB
    nchunk = ew // k
    tail = ew - nchunk * k
    assert tail % _IB == 0

    def body(t_h, src_h, dst_h, z8_h, dp_h,
             d_sp, src_v, dst_v, ts_v, td_v, cb_v, db_v, tb_v):
        c = lax.axis_index("c")
        s = lax.axis_index("s")
        wid = c * _NS + s
        r0 = s * rpt
        lanes = _iota16()
        zeros = jnp.zeros((_L,), jnp.float32)

        @pl.loop(0, rpt // rz)
        def _(m):
            pltpu.sync_copy(z8_h.at[pl.ds(0, rz)],
                            d_sp.at[pl.ds(r0 + m * rz, rz)])

        @pl.loop(0, k // _L)
        def _(g):
            rows = g * _L + lanes
            for j in range(3, 8):
                plsc.store_scatter(cb_v, [rows, _col(j)], zeros)

        plsc.subcore_barrier()

        def chunk(ebase, kc):
            nib = kc // _IB
            pltpu.sync_copy(src_h.at[pl.ds(ebase, kc)], src_v.at[pl.ds(0, kc)])
            pltpu.sync_copy(dst_h.at[pl.ds(ebase, kc)], dst_v.at[pl.ds(0, kc)])

            @pl.loop(0, nib)
            def _(j):
                pltpu.sync_copy(t_h.at[src_v.at[pl.ds(j * _IB, _IB)]],
                                ts_v.at[pl.ds(j * _IB, _IB)])
                pltpu.sync_copy(t_h.at[dst_v.at[pl.ds(j * _IB, _IB)]],
                                td_v.at[pl.ds(j * _IB, _IB)])

            @pl.loop(0, kc // _L)
            def _(g):
                rows = g * _L + lanes
                ts = [plsc.load_gather(ts_v, [rows, _col(j)]) for j in range(12)]
                td = [plsc.load_gather(td_v, [rows, _col(j)]) for j in range(12)]
                d = [td[j] - ts[j] for j in range(3)]
                dist2 = d[0] * d[0] + d[1] * d[1] + d[2] * d[2] + 1e-8
                rcp = 1.0 / dist2
                e = [dj * rcp for dj in d]
                for i in range(3):
                    ci = ((td[3 + 3 * i] - ts[3 + 3 * i]) * e[0] +
                          (td[4 + 3 * i] - ts[4 + 3 * i]) * e[1] +
                          (td[5 + 3 * i] - ts[5 + 3 * i]) * e[2])
                    plsc.store_scatter(cb_v, [rows, _col(i)], ci)

            @pl.loop(0, nib)
            def _(j):
                pltpu.sync_copy(cb_v.at[pl.ds(j * _IB, _IB)],
                                d_sp.at[src_v.at[pl.ds(j * _IB, _IB)]],
                                add=True)

        @pl.loop(0, nchunk)
        def _(ch):
            chunk(wid * ew + ch * k, k)

        if tail:
            chunk(wid * ew + nchunk * k, tail)

        plsc.subcore_barrier()

        @pl.loop(0, rpt // rz)
        def _(m):
            r1 = r0 + m * rz
            pltpu.sync_copy(d_sp.at[pl.ds(r1, rz)], db_v)
            pltpu.sync_copy(t_h.at[pl.ds(r1, rz)], tb_v)

            @pl.loop(0, rz // _L)
            def _(g):
                rows = g * _L + lanes
                degi = plsc.load_gather(tb_v, [rows, _col(12)])
                for j in range(3):
                    v = plsc.load_gather(db_v, [rows, _col(j)]) * degi
                    plsc.store_scatter(db_v, [rows, _col(j)], v)

            pltpu.sync_copy(db_v, dp_h.at[c, pl.ds(r1, rz)])

    return pl.kernel(
        body,
        compiler_params=_CP,
        out_type=jax.ShapeDtypeStruct((_NC, n_nodes_p, 8), jnp.float32),
        mesh=_mesh(),
        scratch_types=[
            pltpu.VMEM_SHARED((n_nodes_p, 8), jnp.float32),
            pltpu.VMEM((k,), jnp.int32),
            pltpu.VMEM((k,), jnp.int32),
            pltpu.VMEM((k, 16), jnp.float32),
            pltpu.VMEM((k, 16), jnp.float32),
            pltpu.VMEM((k, 8), jnp.float32),
            pltpu.VMEM((rz, 8), jnp.float32),
            pltpu.VMEM((rz, 16), jnp.float32),
        ],
        name="mcl_div_pass",
    )(t16, src, dst, zeros8)


def _loss_tc(a, b, c0, c1, n_valid):
    def body(a_ref, b_ref, c0_ref, c1_ref, o_ref):
        lap = c0_ref[...] + c1_ref[...]
        phys = ((_MU / _RHO) * lap + b_ref[...] * (1.0 / _RHO)) * _DT
        res = a_ref[...] - phys
        o_ref[0, 0] = _WEIGHT * jnp.sum(res * res) * (1.0 / n_valid)

    out = pl.pallas_call(
        body,
        out_shape=jax.ShapeDtypeStruct((1, 1), jnp.float32),
        in_specs=[pl.BlockSpec(memory_space=pltpu.VMEM)] * 4,
        out_specs=pl.BlockSpec(memory_space=pltpu.SMEM),
        name="mcl_loss",
    )(a, b, c0, c1)
    return out[0, 0]


def _flatpad(v, n_flat_p):
    v = v.reshape(-1)
    pad = n_flat_p - v.shape[0]
    if pad:
        v = jnp.concatenate([v, jnp.zeros((pad,), v.dtype)])
    return v.reshape(n_flat_p // 128, 128)


def kernel(pred, target, x, pos, edge_index, edge_attr, external_force):
    n = pos.shape[0]
    n_edges = edge_index.shape[1]

    rpt = ((n + _NS - 1) // _NS + 63) // 64 * 64
    n_p = rpt * _NS

    velocity = x[:, 5:8]
    pv8 = jnp.concatenate(
        [pos, velocity, jnp.zeros((n, 2), jnp.float32)], axis=1)
    pv8 = jnp.concatenate([pv8, jnp.zeros((n_p - n, 8), jnp.float32)], axis=0)

    assert n_edges % (_NW * _IB) == 0
    ei = edge_index.astype(jnp.int32)
    src = ei[0]
    dst = ei[1]
    wgt = edge_attr[:, 0].astype(jnp.float32)
    zeros16 = jnp.zeros((_sub_rows(rpt, 2048), 16), jnp.float32)
    zeros8 = jnp.zeros((_sub_rows(rpt, 2048), 8), jnp.float32)

    gp = _pass_a(pv8, src, dst, wgt, zeros16, n_p, n_edges)
    t16 = _combine(pv8, gp, n_p)
    dp = _pass_b(t16, src, dst, zeros8, n_p, n_edges)

    n_flat = 3 * n
    n_flat_p = (n_flat + 127) // 128 * 128
    a = _flatpad(pred[:, 2:5].astype(jnp.float32), n_flat_p)
    b = _flatpad(external_force.astype(jnp.float32), n_flat_p)
    c0 = _flatpad(dp[0, :n, 0:3], n_flat_p)
    c1 = _flatpad(dp[1, :n, 0:3], n_flat_p)
    return _loss_tc(a, b, c0, c1, float(n_flat))
